# baseline (speedup 1.0000x reference)
import functools

import jax
import jax.numpy as jnp
import numpy as np
from jax import lax
from jax.experimental import pallas as pl
from jax.experimental.pallas import tpu as pltpu
from jax.experimental.pallas import tpu_sc as plsc

_D = 256
_NH = 8
_HD = 32
_HALF = 16
_FF = 1024
_NOFF = 9
_S_THETA = 100.0
_L_THETA = 10.0

_OFFS = np.stack(
    np.meshgrid(np.arange(-1, 2), np.arange(-1, 2), indexing="ij"), axis=-1
).reshape(-1, 2).astype(np.float32)


def _freq_tables():
  n_x, n_y = 6, 6
  n_l = _HALF - n_x - n_y
  h = np.arange(_NH, dtype=np.float32)[:, None]
  fx = _S_THETA ** (-(np.arange(n_x, dtype=np.float32)[None, :] + h / _NH) / n_x)
  fy = _S_THETA ** (-(np.arange(n_y, dtype=np.float32)[None, :] + h / _NH) / n_y)
  fl = _L_THETA ** (-(np.arange(n_l, dtype=np.float32)[None, :] + h / _NH) / n_l)
  a0 = np.concatenate([fx, np.zeros((_NH, n_y + n_l), np.float32)], axis=1)
  a1 = np.concatenate(
      [np.zeros((_NH, n_x), np.float32), fy, np.zeros((_NH, n_l), np.float32)],
      axis=1)
  al = np.concatenate([np.zeros((_NH, n_x + n_y), np.float32), fl], axis=1)

  def flat(a):
    return np.concatenate([a, a], axis=1).reshape(1, _NH * _HD)

  return flat(a0), flat(a1), flat(al)


_F0, _F1, _FLT = _freq_tables()
_MLO = np.tile(
    np.concatenate(
        [np.ones((_HALF,), np.float32), np.zeros((_HALF,), np.float32)]),
    _NH).reshape(1, _D)
_MHI = 1.0 - _MLO
_HMAT = np.zeros((_D, _NH), np.float32)
for _h in range(_NH):
  _HMAT[_h * _HD:(_h + 1) * _HD, _h] = 1.0
_HMAT_T = np.ascontiguousarray(_HMAT.T)


def _sc_gather(table, idx):
  rows, d = idx.shape[0], table.shape[1]
  gw = 128
  assert rows % gw == 0
  mesh = plsc.VectorSubcoreMesh(core_axis_name="core",
                                subcore_axis_name="subcore")
  idx2 = idx.reshape(1, rows)

  @functools.partial(
      pl.kernel,
      out_type=jax.ShapeDtypeStruct((rows, d), table.dtype),
      mesh=mesh,
  )
  def gather_kernel(x_hbm, i_hbm, o_hbm):
    def body(i_vmem, o_vmem):
      pltpu.sync_copy(x_hbm.at[i_vmem.at[0]], o_vmem)

    pltpu.emit_pipeline(
        body,
        grid=(rows // gw,),
        in_specs=[pl.BlockSpec((1, gw), index_map=lambda i: (0, i))],
        out_specs=[pl.BlockSpec((gw, d), index_map=lambda i: (i, 0))],
        core_axis_name=("core", "subcore"),
        dimension_semantics=(pltpu.PARALLEL,),
    )(i_hbm, o_hbm)

  return gather_kernel(table, idx2)


def _tc_body(bn, nk, ymax, xmax, x_ref, pos_ref, g_ref, cy_ref, cx_ref,
             lv_ref, f0_ref, f1_ref, flt_ref, qfl_ref, mlo_ref, mhi_ref,
             hm_ref, ht_ref, wq_ref, bq_ref, wk_ref, bk_ref, wv_ref, bv_ref,
             wo_ref, bo_ref, l1s_ref, l1b_ref, l2s_ref, l2b_ref, w1_ref,
             b1_ref, w2_ref, b2_ref, o_ref):
  f32 = jnp.float32
  bf16 = jnp.bfloat16
  x = x_ref[...]
  mu = jnp.mean(x, axis=1, keepdims=True)
  xc = x - mu
  var = jnp.mean(xc * xc, axis=1, keepdims=True)
  xn = xc * lax.rsqrt(var + 1e-5) * l1s_ref[...] + l1b_ref[...]

  mlo = mlo_ref[...]
  mhi = mhi_ref[...]
  f0 = f0_ref[...]
  f1 = f1_ref[...]
  flt = flt_ref[...]

  def sincos(ang):
    magic = 12582912.0
    u = ang * float(1.0 / np.pi)
    big = u + magic
    k = big - magic
    r = u - k
    t = r * float(np.pi)
    t2 = t * t
    sp = t * (1.0 + t2 * (-1.0 / 6.0 + t2 *
                          (1.0 / 120.0 + t2 *
                           (-1.0 / 5040.0 + t2 * (1.0 / 362880.0)))))
    cp = 1.0 + t2 * (-0.5 + t2 *
                     (1.0 / 24.0 + t2 *
                      (-1.0 / 720.0 + t2 *
                       (1.0 / 40320.0 + t2 * (-1.0 / 3628800.0)))))
    par = lax.bitcast_convert_type(big, jnp.int32) & 1
    sign = 1.0 - 2.0 * par.astype(jnp.float32)
    return sp * sign, cp * sign

  def rope(z, ang):
    s, c = sincos(ang)
    zsw = jnp.roll(z, _HALF, axis=1) * mhi - jnp.roll(z, -_HALF, axis=1) * mlo
    return z * c + zsw * s

  qy = pos_ref[:, 0:1]
  qx = pos_ref[:, 1:2]
  qp = jnp.dot(xn.astype(bf16), wq_ref[...],
               preferred_element_type=f32) + bq_ref[...]
  qr = rope(qp, qy * f0 + qx * f1 + qfl_ref[...])

  hm = hm_ref[...]
  ht = ht_ref[...]
  inv_sqrt_hd = float(1.0 / np.sqrt(_HD))

  fb2 = g_ref[...].astype(bf16)
  kl2 = jnp.dot(fb2, wk_ref[...], preferred_element_type=f32) + bk_ref[...]
  vv2 = jnp.dot(fb2, wv_ref[...], preferred_element_type=f32) + bv_ref[...]

  cy3 = cy_ref[...].reshape(1, nk, 1)
  cx3 = cx_ref[...].reshape(1, nk, 1)
  lv3 = lv_ref[...].reshape(1, nk, 1)
  qyf3 = jnp.floor(qy)[:, :, None]
  qxf3 = jnp.floor(qx)[:, :, None]
  ky3 = jnp.clip(qyf3 + cy3, 0.0, ymax)
  kx3 = jnp.clip(qxf3 + cx3, 0.0, xmax)
  ang3 = (ky3 * f0[None] + kx3 * f1[None] + lv3 * flt[None])
  kr2 = rope(kl2, ang3.reshape(bn * nk, _D))

  prod3 = qr[:, None, :] * kr2.reshape(bn, nk, _D)
  sc2 = jnp.dot(prod3.reshape(bn * nk, _D), hm,
                preferred_element_type=f32) * inv_sqrt_hd
  sc3 = sc2.reshape(bn, nk, _NH)
  mx = jnp.max(sc3, axis=1)
  e3 = jnp.exp(sc3 - mx[:, None, :])
  ssum = jnp.sum(e3, axis=1)
  attnw3 = e3 * (1.0 / ssum)[:, None, :]
  attne2 = jnp.dot(attnw3.reshape(bn * nk, _NH), ht,
                   preferred_element_type=f32)
  weighted3 = attne2.reshape(bn, nk, _D) * vv2.reshape(bn, nk, _D)
  attn = jnp.sum(weighted3, axis=1)

  x2 = x_ref[...] + jnp.dot(attn.astype(bf16), wo_ref[...],
                            preferred_element_type=f32) + bo_ref[...]
  mu2 = jnp.mean(x2, axis=1, keepdims=True)
  xc2 = x2 - mu2
  var2 = jnp.mean(xc2 * xc2, axis=1, keepdims=True)
  xn2 = xc2 * lax.rsqrt(var2 + 1e-5) * l2s_ref[...] + l2b_ref[...]
  hdn = jax.nn.gelu(
      jnp.dot(xn2.astype(bf16), w1_ref[...], preferred_element_type=f32) +
      b1_ref[...])
  o_ref[...] = x2 + jnp.dot(hdn.astype(bf16), w2_ref[...],
                            preferred_element_type=f32) + b2_ref[...]


def _fused(x, pos, g3, cy, cx, lv, qfl, wq, bq2, wk, bk2, wv, bv2, wo, bo2,
           l1s, l1b, l2s, l2b, w1, b12, w2, b22, ymax, xmax, nk):
  n = x.shape[0]
  bn = 128
  grid = (n // bn,)
  f32 = jnp.float32

  consts = dict(
      f0=jnp.asarray(_F0), f1=jnp.asarray(_F1), flt=jnp.asarray(_FLT),
      mlo=jnp.asarray(_MLO), mhi=jnp.asarray(_MHI), hmat=jnp.asarray(_HMAT),
      hmat_t=jnp.asarray(_HMAT_T))

  def full_spec(shape):
    nd = len(shape)
    return pl.BlockSpec(shape, lambda i, _nd=nd: (0,) * _nd)

  in_specs = [
      pl.BlockSpec((bn, _D), lambda i: (i, 0)),
      pl.BlockSpec((bn, 2), lambda i: (i, 0)),
      pl.BlockSpec((bn * nk, _D), lambda i: (i, 0)),
      full_spec((nk, 1)),
      full_spec((nk, 1)),
      full_spec((nk, 1)),
      full_spec((1, _D)),
      full_spec((1, _D)),
      full_spec((1, _D)),
      full_spec((1, _D)),
      full_spec((1, _D)),
      full_spec((1, _D)),
      full_spec((_D, _NH)),
      full_spec((_NH, _D)),
      full_spec((_D, _D)),
      full_spec((1, _D)),
      full_spec((_D, _D)),
      full_spec((1, _D)),
      full_spec((_D, _D)),
      full_spec((1, _D)),
      full_spec((_D, _D)),
      full_spec((1, _D)),
      full_spec((1, _D)),
      full_spec((1, _D)),
      full_spec((1, _D)),
      full_spec((1, _D)),
      full_spec((_D, _FF)),
      full_spec((1, _FF)),
      full_spec((_FF, _D)),
      full_spec((1, _D)),
  ]

  body = functools.partial(_tc_body, bn, nk, ymax, xmax)
  return pl.pallas_call(
      body,
      grid=grid,
      in_specs=in_specs,
      out_specs=pl.BlockSpec((bn, _D), lambda i: (i, 0)),
      out_shape=jax.ShapeDtypeStruct((n, _D), f32),
      compiler_params=pltpu.CompilerParams(
          dimension_semantics=("arbitrary",)),
  )(x, pos, g3, cy, cx, lv, consts["f0"], consts["f1"], consts["flt"], qfl,
    consts["mlo"], consts["mhi"], consts["hmat"], consts["hmat_t"], wq, bq2,
    wk, bk2, wv, bv2, wo, bo2, l1s, l1b, l2s, l2b, w1, b12, w2, b22)


def kernel(queries, query_batch_offsets, query_positions, stacked_feature_map,
           level_spatial_shapes, Wq, bq, Wk, bk, Wv, bv, Wo, bo, ln1_s, ln1_b,
           ln2_s, ln2_b, W1, b1, W2, b2):
  b, hm, wm, nl, d = stacked_feature_map.shape
  n = queries.shape[0]
  nk = nl * _NOFF

  lss_f = level_spatial_shapes.astype(jnp.float32)
  max_lvl = jnp.argmax(level_spatial_shapes, axis=0)[0]
  strides = lss_f[max_lvl][None, :] / lss_f
  nq_idx = jnp.arange(n, dtype=query_batch_offsets.dtype)
  batch_idx = jnp.clip(
      jnp.searchsorted(query_batch_offsets, nq_idx, side="right") - 1, 0,
      b - 1).astype(jnp.int32)
  offs = jnp.asarray(_OFFS)
  center = jnp.floor(query_positions)
  cells = (center[:, None, None, :] +
           offs[None, None, :, :] * strides[None, :, None, :])
  yi = jnp.clip(cells[..., 0].astype(jnp.int32), 0, hm - 1)
  xi = jnp.clip(cells[..., 1].astype(jnp.int32), 0, wm - 1)
  lvl = jnp.broadcast_to(
      jnp.arange(nl, dtype=jnp.int32)[None, :, None], (n, nl, _NOFF))
  flat = ((batch_idx[:, None, None] * hm + yi) * wm + xi) * nl + lvl
  idx = flat.reshape(-1)

  table = stacked_feature_map.reshape(b * hm * wm * nl, d)

  cy = (strides[:, 0:1] * offs[:, 0][None, :]).reshape(nk, 1)
  cx = (strides[:, 1:2] * offs[:, 1][None, :]).reshape(nk, 1)
  lv = jnp.asarray(
      np.repeat(np.arange(nl), _NOFF).astype(np.float32).reshape(nk, 1))
  qfl = max_lvl.astype(jnp.float32) * jnp.asarray(_FLT)

  bf16 = jnp.bfloat16
  wargs = (
      Wq.astype(bf16), bq.reshape(1, -1),
      Wk.astype(bf16), bk.reshape(1, -1),
      Wv.astype(bf16), bv.reshape(1, -1),
      Wo.astype(bf16), bo.reshape(1, -1),
      ln1_s.reshape(1, -1), ln1_b.reshape(1, -1),
      ln2_s.reshape(1, -1), ln2_b.reshape(1, -1),
      W1.astype(bf16), b1.reshape(1, -1),
      W2.astype(bf16), b2.reshape(1, -1),
  )

  nchunks = 1
  nc = n // nchunks
  outs = []
  for c in range(nchunks):
    gc = _sc_gather(table, idx[c * nc * nk:(c + 1) * nc * nk])
    outs.append(
        _fused(queries[c * nc:(c + 1) * nc],
               query_positions[c * nc:(c + 1) * nc], gc, cy, cx, lv, qfl,
               *wargs, float(hm - 1), float(wm - 1), nk))
  return jnp.concatenate(outs, axis=0) if nchunks > 1 else outs[0]


# speedup vs baseline: 1.4480x; 1.4480x over previous
import functools

import jax
import jax.numpy as jnp
import numpy as np
from jax import lax
from jax.experimental import pallas as pl
from jax.experimental.pallas import tpu as pltpu
from jax.experimental.pallas import tpu_sc as plsc

_D = 256
_NH = 8
_HD = 32
_HALF = 16
_FF = 1024
_NOFF = 9
_S_THETA = 100.0
_L_THETA = 10.0

_OFFS = np.stack(
    np.meshgrid(np.arange(-1, 2), np.arange(-1, 2), indexing="ij"), axis=-1
).reshape(-1, 2).astype(np.float32)


def _freq_tables():
  n_x, n_y = 6, 6
  n_l = _HALF - n_x - n_y
  h = np.arange(_NH, dtype=np.float32)[:, None]
  fx = _S_THETA ** (-(np.arange(n_x, dtype=np.float32)[None, :] + h / _NH) / n_x)
  fy = _S_THETA ** (-(np.arange(n_y, dtype=np.float32)[None, :] + h / _NH) / n_y)
  fl = _L_THETA ** (-(np.arange(n_l, dtype=np.float32)[None, :] + h / _NH) / n_l)
  a0 = np.concatenate([fx, np.zeros((_NH, n_y + n_l), np.float32)], axis=1)
  a1 = np.concatenate(
      [np.zeros((_NH, n_x), np.float32), fy, np.zeros((_NH, n_l), np.float32)],
      axis=1)
  al = np.concatenate([np.zeros((_NH, n_x + n_y), np.float32), fl], axis=1)

  def flat(a):
    return np.concatenate([a, a], axis=1).reshape(1, _NH * _HD)

  return flat(a0), flat(a1), flat(al)


_F0, _F1, _FLT = _freq_tables()
_MLO = np.tile(
    np.concatenate(
        [np.ones((_HALF,), np.float32), np.zeros((_HALF,), np.float32)]),
    _NH).reshape(1, _D)
_MHI = 1.0 - _MLO
_HMAT = np.zeros((_D, _NH), np.float32)
for _h in range(_NH):
  _HMAT[_h * _HD:(_h + 1) * _HD, _h] = 1.0
_HMAT_T = np.ascontiguousarray(_HMAT.T)


def _sc_gather(table, idx):
  rows, d = idx.shape[0], table.shape[1]
  gw = 128
  assert rows % gw == 0
  mesh = plsc.VectorSubcoreMesh(core_axis_name="core",
                                subcore_axis_name="subcore")
  idx2 = idx.reshape(1, rows)

  @functools.partial(
      pl.kernel,
      out_type=jax.ShapeDtypeStruct((rows, d), table.dtype),
      mesh=mesh,
  )
  def gather_kernel(x_hbm, i_hbm, o_hbm):
    def body(i_vmem, o_vmem):
      pltpu.sync_copy(x_hbm.at[i_vmem.at[0]], o_vmem)

    pltpu.emit_pipeline(
        body,
        grid=(rows // gw,),
        in_specs=[pl.BlockSpec((1, gw), index_map=lambda i: (0, i))],
        out_specs=[pl.BlockSpec((gw, d), index_map=lambda i: (i, 0))],
        core_axis_name=("core", "subcore"),
        dimension_semantics=(pltpu.PARALLEL,),
    )(i_hbm, o_hbm)

  return gather_kernel(table, idx2)


def _tc_body(bn, nk, ymax, xmax, x_ref, pos_ref, g_ref, cy_ref, cx_ref,
             lv_ref, f0_ref, f1_ref, flt_ref, qfl_ref, mlo_ref, mhi_ref,
             hm_ref, ht_ref, wq_ref, bq_ref, wk_ref, bk_ref, wv_ref, bv_ref,
             wo_ref, bo_ref, l1s_ref, l1b_ref, l2s_ref, l2b_ref, w1_ref,
             b1_ref, w2_ref, b2_ref, o_ref):
  f32 = jnp.float32
  bf16 = jnp.bfloat16
  x = x_ref[...]
  mu = jnp.mean(x, axis=1, keepdims=True)
  xc = x - mu
  var = jnp.mean(xc * xc, axis=1, keepdims=True)
  xn = xc * lax.rsqrt(var + 1e-5) * l1s_ref[...] + l1b_ref[...]

  mlo = mlo_ref[...]
  mhi = mhi_ref[...]
  f0 = f0_ref[...]
  f1 = f1_ref[...]
  flt = flt_ref[...]

  def sincos(ang):
    magic = 12582912.0
    u = ang * float(1.0 / np.pi)
    big = u + magic
    k = big - magic
    r = u - k
    t = r * float(np.pi)
    t2 = t * t
    sp = t * (1.0 + t2 * (-1.0 / 6.0 + t2 *
                          (1.0 / 120.0 + t2 *
                           (-1.0 / 5040.0 + t2 * (1.0 / 362880.0)))))
    cp = 1.0 + t2 * (-0.5 + t2 *
                     (1.0 / 24.0 + t2 *
                      (-1.0 / 720.0 + t2 *
                       (1.0 / 40320.0 + t2 * (-1.0 / 3628800.0)))))
    par = lax.bitcast_convert_type(big, jnp.int32) & 1
    sign = 1.0 - 2.0 * par.astype(jnp.float32)
    return sp * sign, cp * sign

  def rope(z, ang):
    s, c = sincos(ang)
    zsw = jnp.roll(z, _HALF, axis=1) * mhi - jnp.roll(z, -_HALF, axis=1) * mlo
    return z * c + zsw * s

  qy = pos_ref[:, 0:1]
  qx = pos_ref[:, 1:2]
  qp = jnp.dot(xn.astype(bf16), wq_ref[...],
               preferred_element_type=f32) + bq_ref[...]
  qr = rope(qp, qy * f0 + qx * f1 + qfl_ref[...])

  hm = hm_ref[...]
  ht = ht_ref[...]
  inv_sqrt_hd = float(1.0 / np.sqrt(_HD))

  fb2 = g_ref[...].reshape(nk * bn, _D).astype(bf16)
  kl2 = jnp.dot(fb2, wk_ref[...], preferred_element_type=f32) + bk_ref[...]
  vv2 = jnp.dot(fb2, wv_ref[...], preferred_element_type=f32) + bv_ref[...]

  cy3 = cy_ref[...].reshape(nk, 1, 1)
  cx3 = cx_ref[...].reshape(nk, 1, 1)
  lv3 = lv_ref[...].reshape(nk, 1, 1)
  qyf3 = jnp.floor(qy)[None, :, :]
  qxf3 = jnp.floor(qx)[None, :, :]
  ky3 = jnp.clip(qyf3 + cy3, 0.0, ymax)
  kx3 = jnp.clip(qxf3 + cx3, 0.0, xmax)
  ang3 = (ky3 * f0[None] + kx3 * f1[None] + lv3 * flt[None])
  kr2 = rope(kl2, ang3.reshape(nk * bn, _D))

  prod3 = qr[None, :, :] * kr2.reshape(nk, bn, _D)
  sc2 = jnp.dot(prod3.reshape(nk * bn, _D), hm,
                preferred_element_type=f32) * inv_sqrt_hd
  sc3 = sc2.reshape(nk, bn, _NH)
  mx = jnp.max(sc3, axis=0)
  e3 = jnp.exp(sc3 - mx[None])
  ssum = jnp.sum(e3, axis=0)
  attnw3 = e3 * (1.0 / ssum)[None]
  attne2 = jnp.dot(attnw3.reshape(nk * bn, _NH), ht,
                   preferred_element_type=f32)
  weighted3 = attne2.reshape(nk, bn, _D) * vv2.reshape(nk, bn, _D)
  attn = jnp.sum(weighted3, axis=0)

  x2 = x_ref[...] + jnp.dot(attn.astype(bf16), wo_ref[...],
                            preferred_element_type=f32) + bo_ref[...]
  mu2 = jnp.mean(x2, axis=1, keepdims=True)
  xc2 = x2 - mu2
  var2 = jnp.mean(xc2 * xc2, axis=1, keepdims=True)
  xn2 = xc2 * lax.rsqrt(var2 + 1e-5) * l2s_ref[...] + l2b_ref[...]
  hdn = jax.nn.gelu(
      jnp.dot(xn2.astype(bf16), w1_ref[...], preferred_element_type=f32) +
      b1_ref[...])
  o_ref[...] = x2 + jnp.dot(hdn.astype(bf16), w2_ref[...],
                            preferred_element_type=f32) + b2_ref[...]


def _fused(x, pos, g3, cy, cx, lv, qfl, wq, bq2, wk, bk2, wv, bv2, wo, bo2,
           l1s, l1b, l2s, l2b, w1, b12, w2, b22, ymax, xmax, nk):
  n = x.shape[0]
  bn = 128
  grid = (n // bn,)
  f32 = jnp.float32

  consts = dict(
      f0=jnp.asarray(_F0), f1=jnp.asarray(_F1), flt=jnp.asarray(_FLT),
      mlo=jnp.asarray(_MLO), mhi=jnp.asarray(_MHI), hmat=jnp.asarray(_HMAT),
      hmat_t=jnp.asarray(_HMAT_T))

  def full_spec(shape):
    nd = len(shape)
    return pl.BlockSpec(shape, lambda i, _nd=nd: (0,) * _nd)

  in_specs = [
      pl.BlockSpec((bn, _D), lambda i: (i, 0)),
      pl.BlockSpec((bn, 2), lambda i: (i, 0)),
      pl.BlockSpec((nk, bn, _D), lambda i: (0, i, 0)),
      full_spec((nk, 1)),
      full_spec((nk, 1)),
      full_spec((nk, 1)),
      full_spec((1, _D)),
      full_spec((1, _D)),
      full_spec((1, _D)),
      full_spec((1, _D)),
      full_spec((1, _D)),
      full_spec((1, _D)),
      full_spec((_D, _NH)),
      full_spec((_NH, _D)),
      full_spec((_D, _D)),
      full_spec((1, _D)),
      full_spec((_D, _D)),
      full_spec((1, _D)),
      full_spec((_D, _D)),
      full_spec((1, _D)),
      full_spec((_D, _D)),
      full_spec((1, _D)),
      full_spec((1, _D)),
      full_spec((1, _D)),
      full_spec((1, _D)),
      full_spec((1, _D)),
      full_spec((_D, _FF)),
      full_spec((1, _FF)),
      full_spec((_FF, _D)),
      full_spec((1, _D)),
  ]

  body = functools.partial(_tc_body, bn, nk, ymax, xmax)
  return pl.pallas_call(
      body,
      grid=grid,
      in_specs=in_specs,
      out_specs=pl.BlockSpec((bn, _D), lambda i: (i, 0)),
      out_shape=jax.ShapeDtypeStruct((n, _D), f32),
      compiler_params=pltpu.CompilerParams(
          dimension_semantics=("arbitrary",)),
  )(x, pos, g3, cy, cx, lv, consts["f0"], consts["f1"], consts["flt"], qfl,
    consts["mlo"], consts["mhi"], consts["hmat"], consts["hmat_t"], wq, bq2,
    wk, bk2, wv, bv2, wo, bo2, l1s, l1b, l2s, l2b, w1, b12, w2, b22)


def kernel(queries, query_batch_offsets, query_positions, stacked_feature_map,
           level_spatial_shapes, Wq, bq, Wk, bk, Wv, bv, Wo, bo, ln1_s, ln1_b,
           ln2_s, ln2_b, W1, b1, W2, b2):
  b, hm, wm, nl, d = stacked_feature_map.shape
  n = queries.shape[0]
  nk = nl * _NOFF

  lss_f = level_spatial_shapes.astype(jnp.float32)
  max_lvl = jnp.argmax(level_spatial_shapes, axis=0)[0]
  strides = lss_f[max_lvl][None, :] / lss_f
  nq_idx = jnp.arange(n, dtype=query_batch_offsets.dtype)
  batch_idx = jnp.clip(
      jnp.searchsorted(query_batch_offsets, nq_idx, side="right") - 1, 0,
      b - 1).astype(jnp.int32)
  offs = jnp.asarray(_OFFS)
  center = jnp.floor(query_positions)
  cells = (center[:, None, None, :] +
           offs[None, None, :, :] * strides[None, :, None, :])
  yi = jnp.clip(cells[..., 0].astype(jnp.int32), 0, hm - 1)
  xi = jnp.clip(cells[..., 1].astype(jnp.int32), 0, wm - 1)
  lvl = jnp.broadcast_to(
      jnp.arange(nl, dtype=jnp.int32)[None, :, None], (n, nl, _NOFF))
  flat = ((batch_idx[:, None, None] * hm + yi) * wm + xi) * nl + lvl
  flat2 = flat.reshape(n, nk)

  table = stacked_feature_map.reshape(b * hm * wm * nl, d)

  cy = (strides[:, 0:1] * offs[:, 0][None, :]).reshape(nk, 1)
  cx = (strides[:, 1:2] * offs[:, 1][None, :]).reshape(nk, 1)
  lv = jnp.asarray(
      np.repeat(np.arange(nl), _NOFF).astype(np.float32).reshape(nk, 1))
  qfl = max_lvl.astype(jnp.float32) * jnp.asarray(_FLT)

  bf16 = jnp.bfloat16
  wargs = (
      Wq.astype(bf16), bq.reshape(1, -1),
      Wk.astype(bf16), bk.reshape(1, -1),
      Wv.astype(bf16), bv.reshape(1, -1),
      Wo.astype(bf16), bo.reshape(1, -1),
      ln1_s.reshape(1, -1), ln1_b.reshape(1, -1),
      ln2_s.reshape(1, -1), ln2_b.reshape(1, -1),
      W1.astype(bf16), b1.reshape(1, -1),
      W2.astype(bf16), b2.reshape(1, -1),
  )

  nchunks = 4 if n % (4 * 128) == 0 else 1
  nc = n // nchunks
  outs = []
  for c in range(nchunks):
    idxc = flat2[c * nc:(c + 1) * nc].T.reshape(-1)
    gc = _sc_gather(table, idxc).reshape(nk, nc, d)
    outs.append(
        _fused(queries[c * nc:(c + 1) * nc],
               query_positions[c * nc:(c + 1) * nc], gc, cy, cx, lv, qfl,
               *wargs, float(hm - 1), float(wm - 1), nk))
  return jnp.concatenate(outs, axis=0) if nchunks > 1 else outs[0]
